# baseline (speedup 1.0000x reference)
import jax
import jax.numpy as jnp
from jax import lax
from jax.experimental import pallas as pl
from jax.experimental.pallas import tpu as pltpu

SEQ_BLOCK = 1024
NBUF = 8


def _add_kernel(x_hbm, w_hbm, o_hbm, xbuf, wbuf, xsem, wsem, osem):
    batch = x_hbm.shape[0]
    nb = x_hbm.shape[1] // SEQ_BLOCK
    nsteps = nb * batch

    def x_slice(t):
        i = t // batch
        b = t % batch
        return (b, pl.ds(i * SEQ_BLOCK, SEQ_BLOCK), slice(None))

    pltpu.make_async_copy(
        w_hbm.at[pl.ds(0, SEQ_BLOCK), :], wbuf.at[0], wsem.at[0]
    ).start()
    for t in range(NBUF):
        pltpu.make_async_copy(
            x_hbm.at[x_slice(t)], xbuf.at[t], xsem.at[t]
        ).start()

    def step(t, carry):
        r = t % NBUF
        i = t // batch
        b = t % batch
        cur = i % 2

        @pl.when(jnp.logical_and(b == 0, i + 1 < nb))
        def _prefetch_w():
            pltpu.make_async_copy(
                w_hbm.at[pl.ds((i + 1) * SEQ_BLOCK, SEQ_BLOCK), :],
                wbuf.at[(i + 1) % 2],
                wsem.at[(i + 1) % 2],
            ).start()

        @pl.when(b == 0)
        def _wait_w():
            pltpu.make_async_copy(
                w_hbm.at[pl.ds(0, SEQ_BLOCK), :], wbuf.at[cur], wsem.at[cur]
            ).wait()

        pltpu.make_async_copy(
            x_hbm.at[x_slice(t)], xbuf.at[r], xsem.at[r]
        ).wait()

        xbuf[r] = xbuf[r] + wbuf[cur]

        pltpu.make_async_copy(
            xbuf.at[r], o_hbm.at[x_slice(t)], osem.at[r]
        ).start()

        @pl.when(t + NBUF < nsteps)
        def _next_load():
            pltpu.make_async_copy(
                xbuf.at[r], o_hbm.at[x_slice(t)], osem.at[r]
            ).wait()
            pltpu.make_async_copy(
                x_hbm.at[x_slice(t + NBUF)], xbuf.at[r], xsem.at[r]
            ).start()

        return carry

    lax.fori_loop(0, nsteps, step, 0)

    for t in range(nsteps - NBUF, nsteps):
        pltpu.make_async_copy(
            xbuf.at[t % NBUF], o_hbm.at[x_slice(t)], osem.at[t % NBUF]
        ).wait()


def kernel(x, embedding_weight):
    batch, seq_len, hidden = x.shape

    return pl.pallas_call(
        _add_kernel,
        in_specs=[
            pl.BlockSpec(memory_space=pltpu.MemorySpace.HBM),
            pl.BlockSpec(memory_space=pltpu.MemorySpace.HBM),
        ],
        out_specs=pl.BlockSpec(memory_space=pltpu.MemorySpace.HBM),
        out_shape=jax.ShapeDtypeStruct(x.shape, x.dtype),
        scratch_shapes=[
            pltpu.VMEM((NBUF, SEQ_BLOCK, hidden), jnp.float32),
            pltpu.VMEM((2, SEQ_BLOCK, hidden), jnp.float32),
            pltpu.SemaphoreType.DMA((NBUF,)),
            pltpu.SemaphoreType.DMA((2,)),
            pltpu.SemaphoreType.DMA((NBUF,)),
        ],
    )(x, embedding_weight)


# speedup vs baseline: 1.0328x; 1.0328x over previous
import jax
import jax.numpy as jnp
from jax import lax
from jax.experimental import pallas as pl
from jax.experimental.pallas import tpu as pltpu

SEQ_BLOCK = 2048
NBUF = 5


def _add_kernel(x_hbm, w_hbm, o_hbm, xbuf, wbuf, xsem, wsem, osem):
    batch = x_hbm.shape[0]
    nb = x_hbm.shape[1] // SEQ_BLOCK
    nsteps = nb * batch

    def x_slice(t):
        i = t // batch
        b = t % batch
        return (b, pl.ds(i * SEQ_BLOCK, SEQ_BLOCK), slice(None))

    pltpu.make_async_copy(
        w_hbm.at[pl.ds(0, SEQ_BLOCK), :], wbuf.at[0], wsem.at[0]
    ).start()
    for t in range(NBUF):
        pltpu.make_async_copy(
            x_hbm.at[x_slice(t)], xbuf.at[t], xsem.at[t]
        ).start()

    def step(t, carry):
        r = t % NBUF
        i = t // batch
        b = t % batch
        cur = i % 2

        @pl.when(jnp.logical_and(b == 0, i + 1 < nb))
        def _prefetch_w():
            pltpu.make_async_copy(
                w_hbm.at[pl.ds((i + 1) * SEQ_BLOCK, SEQ_BLOCK), :],
                wbuf.at[(i + 1) % 2],
                wsem.at[(i + 1) % 2],
            ).start()

        @pl.when(b == 0)
        def _wait_w():
            pltpu.make_async_copy(
                w_hbm.at[pl.ds(0, SEQ_BLOCK), :], wbuf.at[cur], wsem.at[cur]
            ).wait()

        pltpu.make_async_copy(
            x_hbm.at[x_slice(t)], xbuf.at[r], xsem.at[r]
        ).wait()

        xbuf[r] = xbuf[r] + wbuf[cur]

        pltpu.make_async_copy(
            xbuf.at[r], o_hbm.at[x_slice(t)], osem.at[r]
        ).start()

        @pl.when(t + NBUF < nsteps)
        def _next_load():
            pltpu.make_async_copy(
                xbuf.at[r], o_hbm.at[x_slice(t)], osem.at[r]
            ).wait()
            pltpu.make_async_copy(
                x_hbm.at[x_slice(t + NBUF)], xbuf.at[r], xsem.at[r]
            ).start()

        return carry

    lax.fori_loop(0, nsteps, step, 0)

    for t in range(nsteps - NBUF, nsteps):
        pltpu.make_async_copy(
            xbuf.at[t % NBUF], o_hbm.at[x_slice(t)], osem.at[t % NBUF]
        ).wait()


def kernel(x, embedding_weight):
    batch, seq_len, hidden = x.shape

    return pl.pallas_call(
        _add_kernel,
        in_specs=[
            pl.BlockSpec(memory_space=pltpu.MemorySpace.HBM),
            pl.BlockSpec(memory_space=pltpu.MemorySpace.HBM),
        ],
        out_specs=pl.BlockSpec(memory_space=pltpu.MemorySpace.HBM),
        out_shape=jax.ShapeDtypeStruct(x.shape, x.dtype),
        scratch_shapes=[
            pltpu.VMEM((NBUF, SEQ_BLOCK, hidden), jnp.float32),
            pltpu.VMEM((2, SEQ_BLOCK, hidden), jnp.float32),
            pltpu.SemaphoreType.DMA((NBUF,)),
            pltpu.SemaphoreType.DMA((2,)),
            pltpu.SemaphoreType.DMA((NBUF,)),
        ],
    )(x, embedding_weight)


# speedup vs baseline: 1.0348x; 1.0019x over previous
import jax
import jax.numpy as jnp
from jax.experimental import pallas as pl

SEQ_BLOCK = 2048


def _add_kernel(x_ref, w_ref, o_ref):
    o_ref[...] = x_ref[...] + w_ref[...][None, :, :]


def kernel(x, embedding_weight):
    batch, seq_len, hidden = x.shape
    num_blocks = seq_len // SEQ_BLOCK

    return pl.pallas_call(
        _add_kernel,
        grid=(num_blocks, batch),
        in_specs=[
            pl.BlockSpec((1, SEQ_BLOCK, hidden), lambda i, b: (b, i, 0)),
            pl.BlockSpec((SEQ_BLOCK, hidden), lambda i, b: (i, 0)),
        ],
        out_specs=pl.BlockSpec((1, SEQ_BLOCK, hidden), lambda i, b: (b, i, 0)),
        out_shape=jax.ShapeDtypeStruct(x.shape, x.dtype),
    )(x, embedding_weight)
